# baseline (speedup 1.0000x reference)
import functools

import jax
import jax.numpy as jnp
from jax import lax
from jax.experimental import pallas as pl
from jax.experimental.pallas import tpu as pltpu


def _verif_block(f_ref, t_ref, bank_ref, w_ref, b_ref, out0_ref, out1_ref, tgt_ref,
                 *, block_m: int):
    j = pl.program_id(0)
    f = f_ref[...]
    w = w_ref[...]
    bias = b_ref[...]
    t = t_ref[...]
    bank_blk = bank_ref[...]

    B = f.shape[0]
    inv_b = 1.0 / B

    m_ids = j * block_m + lax.broadcasted_iota(jnp.int32, (block_m, B), 0)
    onehot = (m_ids == t).astype(jnp.float32)

    u = bank_blk + lax.dot_general(
        onehot, f * inv_b, (((1,), (0,)), ((), ())),
        preferred_element_type=jnp.float32)

    u2 = u * u
    f2 = f * f
    fw = jnp.concatenate([f * w[:, 0][None, :], f * w[:, 1][None, :]], axis=0)
    cross = lax.dot_general(fw, u, (((1,), (1,)), ((), ())),
                            preferred_element_type=jnp.float32)

    for c, out_ref in ((0, out0_ref), (1, out1_ref)):
        wc = w[:, c][None, :]
        s_c = jnp.sum(u2 * wc, axis=1)
        a_c = jnp.sum(f2 * wc, axis=1) + bias[0, c]
        out_ref[...] = (a_c[:, None] - 2.0 * cross[c * B:(c + 1) * B, :]) \
            + s_c[None, :]

    m_cols = j * block_m + lax.broadcasted_iota(jnp.int32, (B, block_m), 1)
    tgt_ref[...] = (t.reshape(B, 1) == m_cols).astype(jnp.int32)


def kernel(features, targets, bank, W, b):
    B, D = features.shape
    M, _ = bank.shape
    C = W.shape[1]
    block_m = 256
    grid = (M // block_m,)

    t2d = targets.reshape(1, B).astype(jnp.int32)
    b2d = b.reshape(1, C)

    out0, out1, tgt = pl.pallas_call(
        functools.partial(_verif_block, block_m=block_m),
        grid=grid,
        in_specs=[
            pl.BlockSpec((B, D), lambda j: (0, 0)),
            pl.BlockSpec((1, B), lambda j: (0, 0)),
            pl.BlockSpec((block_m, D), lambda j: (j, 0)),
            pl.BlockSpec((D, C), lambda j: (0, 0)),
            pl.BlockSpec((1, C), lambda j: (0, 0)),
        ],
        out_specs=[
            pl.BlockSpec((B, block_m), lambda j: (0, j)),
            pl.BlockSpec((B, block_m), lambda j: (0, j)),
            pl.BlockSpec((B, block_m), lambda j: (0, j)),
        ],
        out_shape=[
            jax.ShapeDtypeStruct((B, M), jnp.float32),
            jax.ShapeDtypeStruct((B, M), jnp.float32),
            jax.ShapeDtypeStruct((B, M), jnp.int32),
        ],
        compiler_params=pltpu.CompilerParams(
            dimension_semantics=("parallel",)),
    )(features, t2d, bank, W, b2d)

    bank_outputs = jnp.stack([out0, out1], axis=-1).reshape(B * M, C)
    bank_targets = tgt.reshape(-1)
    return bank_outputs, bank_targets


# speedup vs baseline: 1.3572x; 1.3572x over previous
import functools

import jax
import jax.numpy as jnp
from jax import lax
from jax.experimental import pallas as pl
from jax.experimental.pallas import tpu as pltpu


def _dot_t(a, b):
    return lax.dot_general(a, b, (((1,), (1,)), ((), ())),
                           preferred_element_type=jnp.float32)


def _verif_block(f_ref, t_ref, bank_ref, wt_ref, b_ref,
                 out0_ref, out1_ref, tgt_ref,
                 fw_scr, ffx_scr, a_scr,
                 *, block_m: int):
    j = pl.program_id(0)
    B = f_ref.shape[0]
    inv_b = 1.0 / B
    t = t_ref[...]

    @pl.when(j == 0)
    def _prologue():
        f = f_ref[...]
        wt = wt_ref[...]
        bias = b_ref[...]
        fb = f * inv_b
        fw = jnp.concatenate([f * wt[0, :][None, :], f * wt[1, :][None, :]],
                             axis=0)
        fw_scr[...] = fw
        tc = t.reshape(B, 1)
        p = (tc == t).astype(jnp.float32)
        rows = lax.broadcasted_iota(jnp.int32, (B, B), 0)
        cols = lax.broadcasted_iota(jnp.int32, (B, B), 1)
        before = jnp.where(cols < rows, p, 0.0)
        rep = (jnp.sum(before, axis=1) == 0.0).astype(jnp.float32)
        dm = lax.dot_general(p, fb, (((1,), (0,)), ((), ())),
                             preferred_element_type=jnp.float32)
        ffd = _dot_t(fw, dm)
        dm2 = dm * dm
        rc2_0 = jnp.sum(dm2 * wt[0, :][None, :], axis=1)
        rc2_1 = jnp.sum(dm2 * wt[1, :][None, :], axis=1)
        ffx = jnp.concatenate(
            [ffd, rc2_0[None, :], rc2_1[None, :],
             jnp.zeros((6, B), jnp.float32)], axis=0)
        ffx_scr[...] = ffx * rep[None, :]
        f2 = f * f
        a0 = jnp.sum(f2 * wt[0, :][None, :], axis=1) + bias[0, 0]
        a1 = jnp.sum(f2 * wt[1, :][None, :], axis=1) + bias[0, 1]
        a_scr[...] = jnp.concatenate(
            [a0[None, :], a1[None, :], jnp.zeros((6, B), jnp.float32)], axis=0)

    bank_blk = bank_ref[...]
    wt = wt_ref[...]
    fw = fw_scr[...]
    bank2 = bank_blk * bank_blk

    cross_b = _dot_t(fw, bank_blk)
    s_b = _dot_t(wt, bank2)
    m_cols = j * block_m + lax.broadcasted_iota(jnp.int32, (B, block_m), 1)
    o_blk = (t.reshape(B, 1) == m_cols)
    of = o_blk.astype(jnp.float32)
    cx = lax.dot_general(ffx_scr[...], of, (((1,), (0,)), ((), ())),
                         preferred_element_type=jnp.float32)

    a = a_scr[...]
    two_inv_b = 2.0 * inv_b
    for c, out_ref in ((0, out0_ref), (1, out1_ref)):
        cb_c = cross_b[c * B:(c + 1) * B, :]
        sx1_c = two_inv_b * jnp.sum(of * cb_c, axis=0)
        s_c = s_b[c, :] + cx[2 * B + c, :] + sx1_c
        out_ref[...] = (a[c, :][:, None]
                        - 2.0 * (cb_c + cx[c * B:(c + 1) * B, :])
                        + s_c[None, :])

    tgt_ref[...] = o_blk.astype(jnp.int32)


def kernel(features, targets, bank, W, b):
    B, D = features.shape
    M, _ = bank.shape
    C = W.shape[1]
    block_m = 256
    grid = (M // block_m,)

    t2d = targets.reshape(1, B).astype(jnp.int32)
    wt = W.T
    b2d = b.reshape(1, C)

    out0, out1, tgt = pl.pallas_call(
        functools.partial(_verif_block, block_m=block_m),
        grid=grid,
        in_specs=[
            pl.BlockSpec((B, D), lambda j: (0, 0)),
            pl.BlockSpec((1, B), lambda j: (0, 0)),
            pl.BlockSpec((block_m, D), lambda j: (j, 0)),
            pl.BlockSpec((C, D), lambda j: (0, 0)),
            pl.BlockSpec((1, C), lambda j: (0, 0)),
        ],
        out_specs=[
            pl.BlockSpec((B, block_m), lambda j: (0, j)),
            pl.BlockSpec((B, block_m), lambda j: (0, j)),
            pl.BlockSpec((B, block_m), lambda j: (0, j)),
        ],
        out_shape=[
            jax.ShapeDtypeStruct((B, M), jnp.float32),
            jax.ShapeDtypeStruct((B, M), jnp.float32),
            jax.ShapeDtypeStruct((B, M), jnp.int32),
        ],
        scratch_shapes=[
            pltpu.VMEM((2 * B, D), jnp.float32),
            pltpu.VMEM((2 * B + 8, B), jnp.float32),
            pltpu.VMEM((8, B), jnp.float32),
        ],
        compiler_params=pltpu.CompilerParams(
            dimension_semantics=("arbitrary",)),
    )(features, t2d, bank, wt, b2d)

    bank_outputs = jnp.stack([out0, out1], axis=-1).reshape(B * M, C)
    bank_targets = tgt.reshape(-1)
    return bank_outputs, bank_targets
